# baseline (speedup 1.0000x reference)
import functools

import jax
import jax.numpy as jnp
from jax import lax
from jax.experimental import pallas as pl
from jax.experimental.pallas import tpu as pltpu
from jax.experimental.pallas import tpu_sc as plsc

NS = 16
L = 16
OWN_BITS = 16
OWN = 1 << OWN_BITS


def _dyemb_sc(batch, dimp):
    rows_t = batch // NS
    nvec = batch // L

    mesh = plsc.VectorSubcoreMesh(
        core_axis_name="c", subcore_axis_name="s", num_cores=1)

    @functools.partial(
        pl.kernel,
        out_type=jax.ShapeDtypeStruct((batch, dimp), jnp.float32),
        mesh=mesh,
        compiler_params=pltpu.CompilerParams(
            needs_layout_passes=False, use_tc_tiling_on_sc=True),
        scratch_types=[
            pltpu.HBM((NS * OWN,), jnp.int32),
            pltpu.VMEM((batch,), jnp.int32),
            pltpu.VMEM((OWN,), jnp.int32),
            pltpu.VMEM((rows_t,), jnp.int32),
            pltpu.VMEM((2, 128, dimp), jnp.float32),
            pltpu.SemaphoreType.DMA,
        ],
    )
    def k(idx_hbm, values_hbm, out_hbm, p_tab, idx_v, tab_v, p_v, rows_v, sem):
        tid = lax.axis_index("s")
        lane = lax.iota(jnp.int32, L)

        pltpu.sync_copy(idx_hbm, idx_v)

        def scan_step(i, carry):
            start = pl.multiple_of(i * L, L)
            x = idx_v[pl.ds(start, L)]
            mine = lax.shift_right_logical(x, OWN_BITS) == tid
            _, last = plsc.scan_count(x)
            xl = x & (OWN - 1)
            pos = i * L + lane
            plsc.store_scatter(tab_v, [xl], pos, mask=last & mine)
            return carry

        lax.fori_loop(0, nvec, scan_step, 0, unroll=4)

        pltpu.sync_copy(tab_v, p_tab.at[pl.ds(tid * OWN, OWN)])
        plsc.subcore_barrier()

        tbase = tid * rows_t
        cps = [
            pltpu.async_copy(
                p_tab.at[idx_v.at[pl.ds(tbase + c * 128, 128)]],
                p_v.at[pl.ds(c * 128, 128)], sem)
            for c in range(rows_t // 128)
        ]
        for cp in cps:
            cp.wait()

        def row_gather(c, buf):
            return pltpu.async_copy(
                values_hbm.at[p_v.at[pl.ds(c * 128, 128)]],
                rows_v.at[buf], sem)
        rchunks = rows_t // 128
        pend = row_gather(0, 0)
        for c in range(rchunks):
            pend.wait()
            if c + 1 < rchunks:
                nxt = row_gather(c + 1, (c + 1) % 2)
            pltpu.sync_copy(rows_v.at[c % 2],
                            out_hbm.at[pl.ds(tbase + c * 128, 128)])
            if c + 1 < rchunks:
                pend = nxt

    return k


@jax.jit
def kernel(raw_feature, node_idxs, values):
    del raw_feature
    batch, dim = values.shape
    values128 = jnp.pad(values, ((0, 0), (0, 128 - dim)))
    out128 = _dyemb_sc(batch, 128)(node_idxs.astype(jnp.int32), values128)
    return out128[:, :dim]
